# baseline (speedup 1.0000x reference)
import functools

import jax
import jax.numpy as jnp
from jax import lax
from jax.experimental import pallas as pl
from jax.experimental.pallas import tpu as pltpu
from jax.experimental.pallas import tpu_sc as plsc

_NC = 2
_NS = 16
_NW = _NC * _NS

_NP = 50176
_NODES_PER_CHUNK = 8


def _elu(x):
    return jnp.where(x > 0, x, jnp.exp(jnp.minimum(x, 0.0)) - 1.0)


@functools.lru_cache(maxsize=None)
def _make_gather(F, SP):
    CK = _NODES_PER_CHUNK * SP
    n_chunks = _NP // _NODES_PER_CHUNK
    ch_per_w = n_chunks // _NW
    nbuf = 4
    rounds = ch_per_w // nbuf
    assert ch_per_w % nbuf == 0 and CK <= 128

    mesh = plsc.VectorSubcoreMesh(core_axis_name="c", subcore_axis_name="s")

    @functools.partial(
        pl.kernel,
        out_type=jax.ShapeDtypeStruct((_NP // _NODES_PER_CHUNK, CK, F), jnp.float32),
        mesh=mesh,
        scratch_types=[
            pltpu.VMEM((ch_per_w * CK,), jnp.int32),
            pltpu.VMEM((nbuf, CK, F), jnp.float32),
            [pltpu.SemaphoreType.DMA] * nbuf,
            [pltpu.SemaphoreType.DMA] * nbuf,
        ],
        compiler_params=pltpu.CompilerParams(use_tc_tiling_on_sc=False),
    )
    def gk(idx_hbm, table_hbm, out_hbm, idx_v, bufs, sg, sw):
        wid = lax.axis_index("s") * _NC + lax.axis_index("c")
        c0 = wid * ch_per_w
        pltpu.sync_copy(idx_hbm.at[pl.ds(c0 * CK, ch_per_w * CK)], idx_v)

        def gather_start(jj, b):
            pltpu.make_async_copy(
                table_hbm.at[idx_v.at[pl.ds(jj * CK, CK)]], bufs.at[b], sg[b]
            ).start()

        def gather_wait(b):
            pltpu.make_async_copy(
                table_hbm.at[idx_v.at[pl.ds(0, CK)]], bufs.at[b], sg[b]
            ).wait()

        def write_start(jj, b):
            pltpu.make_async_copy(bufs.at[b], out_hbm.at[c0 + jj], sw[b]).start()

        def write_wait(b):
            pltpu.make_async_copy(bufs.at[b], out_hbm.at[c0], sw[b]).wait()

        for b in range(nbuf):
            gather_start(b, b)

        def round_body(r, carry):
            for b in range(nbuf):
                jj = r * nbuf + b
                gather_wait(b)
                write_start(jj, b)

                @pl.when(r < rounds - 1)
                def _():
                    write_wait(b)
                    gather_start(jj + nbuf, b)

            return carry

        lax.fori_loop(0, rounds, round_body, 0)
        for b in range(nbuf):
            write_wait(b)

    return gk


@functools.lru_cache(maxsize=None)
def _make_mm_elu(N, K, Fout, BN):
    def body(g_ref, w_ref, b_ref, o_ref):
        acc = jnp.dot(g_ref[...], w_ref[...], preferred_element_type=jnp.float32)
        o_ref[...] = _elu(acc + b_ref[...])

    return pl.pallas_call(
        body,
        grid=(pl.cdiv(N, BN),),
        in_specs=[
            pl.BlockSpec((BN, K), lambda i: (i, 0)),
            pl.BlockSpec((K, Fout), lambda i: (0, 0)),
            pl.BlockSpec((1, Fout), lambda i: (0, 0)),
        ],
        out_specs=pl.BlockSpec((BN, Fout), lambda i: (i, 0)),
        out_shape=jax.ShapeDtypeStruct((N, Fout), jnp.float32),
    )


@functools.lru_cache(maxsize=None)
def _make_mm_elu_kblocks(NCH, T, Fout, BC):
    BN = 8 * BC

    def body(g_ref, w_ref, b_ref, o_ref):
        acc = b_ref[...].astype(jnp.float32)
        for t in range(T):
            lhs = jnp.reshape(g_ref[:, t], (BN, 128))
            acc = acc + jnp.dot(
                lhs, w_ref[t], preferred_element_type=jnp.float32
            )
        o_ref[...] = _elu(acc)

    return pl.pallas_call(
        body,
        grid=(NCH // BC,),
        in_specs=[
            pl.BlockSpec((BC, T, 8, 128), lambda i: (i, 0, 0, 0)),
            pl.BlockSpec((T, 128, Fout), lambda i: (0, 0, 0)),
            pl.BlockSpec((1, Fout), lambda i: (0, 0)),
        ],
        out_specs=pl.BlockSpec((BN, Fout), lambda i: (i, 0)),
        out_shape=jax.ShapeDtypeStruct((NCH * 8, Fout), jnp.float32),
    )


@functools.lru_cache(maxsize=None)
def _make_head(N, NOUT, F3, F4, C, BN):
    def body(x_ref, w4_ref, b4_ref, w5_ref, b5_ref, o_ref):
        h = _elu(
            jnp.dot(x_ref[...], w4_ref[...], preferred_element_type=jnp.float32)
            + b4_ref[...]
        )
        z = (
            jnp.dot(h, w5_ref[...], preferred_element_type=jnp.float32)
            + b5_ref[...]
        )
        m = jnp.max(z, axis=1, keepdims=True)
        lse = jnp.log(jnp.sum(jnp.exp(z - m), axis=1, keepdims=True)) + m
        o_ref[...] = z - lse

    return pl.pallas_call(
        body,
        grid=(pl.cdiv(NOUT, BN),),
        in_specs=[
            pl.BlockSpec((BN, F3), lambda i: (i, 0)),
            pl.BlockSpec((F3, F4), lambda i: (0, 0)),
            pl.BlockSpec((1, F4), lambda i: (0, 0)),
            pl.BlockSpec((F4, C), lambda i: (0, 0)),
            pl.BlockSpec((1, C), lambda i: (0, 0)),
        ],
        out_specs=pl.BlockSpec((BN, C), lambda i: (i, 0)),
        out_shape=jax.ShapeDtypeStruct((NOUT, C), jnp.float32),
    )


def kernel(pos, indices, W0, b0, W1, b1, W2, b2, W3, b3, W4, b4, W5, b5):
    n, s = indices.shape

    sp_by_f = {16: 16, 32: 12, 64: 10}
    nch = _NP // _NODES_PER_CHUNK

    idx_pad = {}
    for f in (16, 32, 64):
        sp = sp_by_f[f]
        t_blocks = sp * f // 128
        own = jnp.broadcast_to(
            jnp.arange(_NP, dtype=jnp.int32)[:, None], (_NP, sp)
        )
        ind_pad = jnp.pad(indices, ((0, _NP - n), (0, sp - s)))
        real = (
            jnp.arange(_NP, dtype=jnp.int32)[:, None] < n
        ) & (jnp.arange(sp, dtype=jnp.int32)[None, :] < s)
        ip = jnp.where(real, ind_pad, own)
        ip = ip.reshape(nch, _NODES_PER_CHUNK, t_blocks, sp // t_blocks)
        idx_pad[f] = ip.transpose(0, 2, 1, 3).reshape(-1)

    pos16 = jnp.pad(pos, ((0, _NP - n), (0, 16 - pos.shape[1])))
    W0p = jnp.pad(W0, ((0, 16 - W0.shape[0]), (0, 0)))

    x = _make_mm_elu(_NP, 16, W0.shape[1], 1024)(pos16, W0p, b0.reshape(1, -1))
    for W, b in ((W1, b1), (W2, b2), (W3, b3)):
        f = x.shape[1]
        sp = sp_by_f[f]
        t_blocks = sp * f // 128
        fout = W.shape[1]
        Wp = jnp.pad(W, ((0, (sp - s) * f), (0, 0))).reshape(t_blocks, 128, fout)
        g = _make_gather(f, sp)(idx_pad[f], x)
        x = _make_mm_elu_kblocks(nch, t_blocks, fout, 128)(
            g.reshape(nch, t_blocks, 8, 128), Wp, b.reshape(1, -1)
        )

    return _make_head(_NP, n, W4.shape[0], W4.shape[1], W5.shape[1], 1024)(
        x, W4, b4.reshape(1, -1), W5, b5.reshape(1, -1)
    )


# speedup vs baseline: 2.4130x; 2.4130x over previous
import functools

import jax
import jax.numpy as jnp
from jax import lax
from jax.experimental import pallas as pl
from jax.experimental.pallas import tpu as pltpu
from jax.experimental.pallas import tpu_sc as plsc

_NC = 2
_NS = 16
_NW = _NC * _NS

_NP = 50176
_NODES_PER_CHUNK = 8


def _elu(x):
    return jnp.where(x > 0, x, jnp.exp(jnp.minimum(x, 0.0)) - 1.0)


@functools.lru_cache(maxsize=None)
def _make_gather(F, SP):
    CK = _NODES_PER_CHUNK * SP
    n_chunks = _NP // _NODES_PER_CHUNK
    ch_per_w = n_chunks // _NW
    nbuf = 4
    rounds = ch_per_w // nbuf
    assert ch_per_w % nbuf == 0 and CK <= 128

    mesh = plsc.VectorSubcoreMesh(core_axis_name="c", subcore_axis_name="s")

    @functools.partial(
        pl.kernel,
        out_type=jax.ShapeDtypeStruct((_NP // _NODES_PER_CHUNK, CK, F), jnp.float32),
        mesh=mesh,
        scratch_types=[
            pltpu.VMEM((ch_per_w * CK,), jnp.int32),
            pltpu.VMEM((nbuf, CK, F), jnp.float32),
            [pltpu.SemaphoreType.DMA] * nbuf,
            [pltpu.SemaphoreType.DMA] * nbuf,
        ],
        compiler_params=pltpu.CompilerParams(use_tc_tiling_on_sc=False),
    )
    def gk(idx_hbm, table_hbm, out_hbm, idx_v, bufs, sg, sw):
        wid = lax.axis_index("s") * _NC + lax.axis_index("c")
        c0 = wid * ch_per_w
        pltpu.sync_copy(idx_hbm.at[pl.ds(c0 * CK, ch_per_w * CK)], idx_v)

        def gather_start(jj, b):
            pltpu.make_async_copy(
                table_hbm.at[idx_v.at[pl.ds(jj * CK, CK)]], bufs.at[b], sg[b]
            ).start()

        def gather_wait(b):
            pltpu.make_async_copy(
                table_hbm.at[idx_v.at[pl.ds(0, CK)]], bufs.at[b], sg[b]
            ).wait()

        def write_start(jj, b):
            pltpu.make_async_copy(bufs.at[b], out_hbm.at[c0 + jj], sw[b]).start()

        def write_wait(b):
            pltpu.make_async_copy(bufs.at[b], out_hbm.at[c0], sw[b]).wait()

        for b in range(nbuf):
            gather_start(b, b)

        def round_body(r, carry):
            for b in range(nbuf):
                jj = r * nbuf + b
                gather_wait(b)
                write_start(jj, b)

                @pl.when(r < rounds - 1)
                def _():
                    write_wait(b)
                    gather_start(jj + nbuf, b)

            return carry

        lax.fori_loop(0, rounds, round_body, 0)
        for b in range(nbuf):
            write_wait(b)

    return gk


@functools.lru_cache(maxsize=None)
def _make_mm_elu(N, K, Fout, BN):
    def body(g_ref, w_ref, b_ref, o_ref):
        acc = jnp.dot(g_ref[...], w_ref[...], preferred_element_type=jnp.float32)
        o_ref[...] = _elu(acc + b_ref[...])

    return pl.pallas_call(
        body,
        grid=(pl.cdiv(N, BN),),
        in_specs=[
            pl.BlockSpec((BN, K), lambda i: (i, 0)),
            pl.BlockSpec((K, Fout), lambda i: (0, 0)),
            pl.BlockSpec((1, Fout), lambda i: (0, 0)),
        ],
        out_specs=pl.BlockSpec((BN, Fout), lambda i: (i, 0)),
        out_shape=jax.ShapeDtypeStruct((N, Fout), jnp.float32),
    )


@functools.lru_cache(maxsize=None)
def _make_mm_elu_kblocks(NCH, T, Fout, BC):
    BN = 8 * BC

    def body(g_ref, w_ref, b_ref, o_ref):
        acc = b_ref[...].astype(jnp.float32)
        for t in range(T):
            lhs = jnp.reshape(g_ref[:, t], (BN, 128))
            acc = acc + jnp.dot(
                lhs, w_ref[t], preferred_element_type=jnp.float32
            )
        o_ref[...] = _elu(acc)

    return pl.pallas_call(
        body,
        grid=(NCH // BC,),
        in_specs=[
            pl.BlockSpec((BC, T, 8, 128), lambda i: (i, 0, 0, 0)),
            pl.BlockSpec((T, 128, Fout), lambda i: (0, 0, 0)),
            pl.BlockSpec((1, Fout), lambda i: (0, 0)),
        ],
        out_specs=pl.BlockSpec((BN, Fout), lambda i: (i, 0)),
        out_shape=jax.ShapeDtypeStruct((NCH * 8, Fout), jnp.float32),
    )


@functools.lru_cache(maxsize=None)
def _make_head(N, NOUT, F3, F4, C, BN):
    def body(x_ref, w4_ref, b4_ref, w5_ref, b5_ref, o_ref):
        h = _elu(
            jnp.dot(x_ref[...], w4_ref[...], preferred_element_type=jnp.float32)
            + b4_ref[...]
        )
        z = (
            jnp.dot(h, w5_ref[...], preferred_element_type=jnp.float32)
            + b5_ref[...]
        )
        m = jnp.max(z, axis=1, keepdims=True)
        lse = jnp.log(jnp.sum(jnp.exp(z - m), axis=1, keepdims=True)) + m
        o_ref[...] = z - lse

    return pl.pallas_call(
        body,
        grid=(pl.cdiv(NOUT, BN),),
        in_specs=[
            pl.BlockSpec((BN, F3), lambda i: (i, 0)),
            pl.BlockSpec((F3, F4), lambda i: (0, 0)),
            pl.BlockSpec((1, F4), lambda i: (0, 0)),
            pl.BlockSpec((F4, C), lambda i: (0, 0)),
            pl.BlockSpec((1, C), lambda i: (0, 0)),
        ],
        out_specs=pl.BlockSpec((BN, C), lambda i: (i, 0)),
        out_shape=jax.ShapeDtypeStruct((NOUT, C), jnp.float32),
    )


def kernel(pos, indices, W0, b0, W1, b1, W2, b2, W3, b3, W4, b4, W5, b5):
    n, s = indices.shape

    sp_by_f = {16: 16, 32: 12, 64: 10}
    nch = _NP // _NODES_PER_CHUNK

    idx_pad = {}
    for f in (16, 32, 64):
        sp = sp_by_f[f]
        t_blocks = sp * f // 128
        own = jnp.broadcast_to(
            jnp.arange(_NP, dtype=jnp.int32)[:, None], (_NP, sp)
        )
        ind_pad = jnp.pad(indices, ((0, _NP - n), (0, sp - s)))
        real = (
            jnp.arange(_NP, dtype=jnp.int32)[:, None] < n
        ) & (jnp.arange(sp, dtype=jnp.int32)[None, :] < s)
        ip = jnp.where(real, ind_pad, own)
        idx_pad[f] = ip.reshape(-1)

    pos16 = jnp.pad(pos, ((0, _NP - n), (0, 16 - pos.shape[1])))
    W0p = jnp.pad(W0, ((0, 16 - W0.shape[0]), (0, 0)))

    x = _make_mm_elu(_NP, 16, W0.shape[1], 1024)(pos16, W0p, b0.reshape(1, -1))
    for W, b in ((W1, b1), (W2, b2), (W3, b3)):
        f = x.shape[1]
        sp = sp_by_f[f]
        fout = W.shape[1]
        Wp = jnp.pad(W, ((0, (sp - s) * f), (0, 0)))
        g = _make_gather(f, sp)(idx_pad[f], x)
        x = _make_mm_elu(_NP, sp * f, fout, 1024)(
            g.reshape(_NP, sp * f), Wp, b.reshape(1, -1)
        )

    return _make_head(_NP, n, W4.shape[0], W4.shape[1], W5.shape[1], 1024)(
        x, W4, b4.reshape(1, -1), W5, b5.reshape(1, -1)
    )
